# Initial kernel scaffold; baseline (speedup 1.0000x reference)
import jax
import jax.numpy as jnp
from jax.experimental import pallas as pl


def kernel(packed_tokens, cu_seq_lens, table, W1, b1, W2, b2):
    raise NotImplementedError("write your pallas kernel here")


# speedup vs baseline: 21.7034x; 21.7034x over previous
import jax
import jax.numpy as jnp
from jax.experimental import pallas as pl

TOTAL = 32768
NSEG = 16
VOCAB = 17
VPAD = 32
D = 512
CHUNK = 4096
NCHUNK = TOTAL // CHUNK


def _hist_body(lo_ref, hi_ref, tok_ref, hist_ref):
    step = pl.program_id(0)
    base = step * CHUNK
    tok = tok_ref[...]
    voc = jax.lax.broadcasted_iota(jnp.int32, (1, VPAD), 1)
    toh = (tok == voc).astype(jnp.float32)
    pos = base + jax.lax.broadcasted_iota(jnp.int32, (NSEG, CHUNK), 1)
    seg_mask = jnp.logical_and(pos >= lo_ref[...], pos < hi_ref[...])
    m = seg_mask.astype(jnp.float32)
    part = jax.lax.dot_general(
        m, toh, (((1,), (0,)), ((), ())), preferred_element_type=jnp.float32
    )

    @pl.when(step == 0)
    def _():
        hist_ref[...] = part

    @pl.when(step != 0)
    def _():
        hist_ref[...] += part


def _mlp_body(hist_ref, tab_ref, w1_ref, b1_ref, w2_ref, b2_ref, out_ref):
    hist = hist_ref[...]
    counts = jnp.sum(hist, axis=1, keepdims=True)
    hn = hist / counts
    h = jnp.dot(tab_ref[...], w1_ref[...], preferred_element_type=jnp.float32)
    h = h + b1_ref[...]
    g = 0.5 * h * (1.0 + jax.lax.erf(h * 0.7071067811865476))
    mo = jnp.dot(g, w2_ref[...], preferred_element_type=jnp.float32)
    mo = mo + b2_ref[...]
    out_ref[...] = jnp.dot(hn, mo, preferred_element_type=jnp.float32)


def kernel(packed_tokens, cu_seq_lens, table, W1, b1, W2, b2):
    tok2d = packed_tokens.reshape(TOTAL, 1)
    cu = cu_seq_lens.astype(jnp.int32)
    lo = cu[:NSEG].reshape(NSEG, 1)
    hi = cu[1:].reshape(NSEG, 1)
    tab = jnp.zeros((VPAD, D), jnp.float32).at[:VOCAB].set(table)

    hist = pl.pallas_call(
        _hist_body,
        grid=(NCHUNK,),
        in_specs=[
            pl.BlockSpec((NSEG, 1), lambda i: (0, 0)),
            pl.BlockSpec((NSEG, 1), lambda i: (0, 0)),
            pl.BlockSpec((CHUNK, 1), lambda i: (i, 0)),
        ],
        out_specs=pl.BlockSpec((NSEG, VPAD), lambda i: (0, 0)),
        out_shape=jax.ShapeDtypeStruct((NSEG, VPAD), jnp.float32),
    )(lo, hi, tok2d)

    out = pl.pallas_call(
        _mlp_body,
        out_shape=jax.ShapeDtypeStruct((NSEG, D), jnp.float32),
    )(hist, tab, W1.T, b1.reshape(1, D), W2.T, b2.reshape(1, D))
    return out
